# baseline (speedup 1.0000x reference)
import jax
import jax.numpy as jnp
from jax.experimental import pallas as pl


def _copy_body(w_ref, o_ref):
    o_ref[...] = w_ref[: o_ref.shape[0], :]


def kernel(patch, W):
    n = patch.shape[1] + 1
    d = W.shape[1]
    out = pl.pallas_call(
        _copy_body,
        out_shape=jax.ShapeDtypeStruct((n, d), W.dtype),
    )(W)
    return out[None]


# speedup vs baseline: 1.0301x; 1.0301x over previous
import jax
import jax.numpy as jnp
from jax.experimental import pallas as pl
from jax.experimental.pallas import tpu as pltpu


def kernel(patch, W):
    n = patch.shape[1] + 1
    d = W.shape[1]
    n_pad = (n + 7) // 8 * 8

    def body(w_ref, o_ref):
        o_ref[...] = w_ref[:n, :]

    out = pl.pallas_call(
        body,
        out_shape=jax.ShapeDtypeStruct((n, d), W.dtype),
        grid=(1,),
        in_specs=[pl.BlockSpec((n_pad, d), lambda i: (0, 0))],
        out_specs=pl.BlockSpec((n, d), lambda i: (0, 0)),
    )(W)
    return out[None]


# speedup vs baseline: 1.0658x; 1.0347x over previous
import jax
import jax.numpy as jnp
from jax.experimental import pallas as pl
from jax.experimental.pallas import tpu as pltpu

_BLOCK = 296


def kernel(patch, W):
    n = patch.shape[1] + 1
    d = W.shape[1]
    steps = (n + _BLOCK - 1) // _BLOCK

    def body(w_ref, o_ref):
        o_ref[0, ...] = w_ref[...]

    out = pl.pallas_call(
        body,
        out_shape=jax.ShapeDtypeStruct((1, n, d), W.dtype),
        grid=(steps,),
        in_specs=[pl.BlockSpec((_BLOCK, d), lambda i: (i, 0))],
        out_specs=pl.BlockSpec((1, _BLOCK, d), lambda i: (0, i, 0)),
    )(W)
    return out
